# baseline (speedup 1.0000x reference)
import functools

import jax
import jax.numpy as jnp
from jax import lax
from jax.experimental import pallas as pl
from jax.experimental.pallas import tpu as pltpu
from jax.experimental.pallas import tpu_sc as plsc

N = 16384
P = 4096
NBLK = 1024
NSTEPS = N // NBLK
CHUNK = 128
CPB = NBLK // CHUNK

NW = 32
PPW = P // NW
LANES = 16
GROUPS = PPW // LANES


def _round_bf16(x):
    b = lax.bitcast_convert_type(x, jnp.int32)
    r = (b + jnp.int32(0x7FFF) + ((b >> 16) & jnp.int32(1))) & jnp.int32(
        ~0xFFFF)
    return lax.bitcast_convert_type(r, jnp.float32)


def _argmin_body(mu_ref, cp_ref, out_ref, rmin_ref, ridx_ref):
    j = pl.program_id(0)

    @pl.when(j == 0)
    def _init():
        rmin_ref[...] = jnp.full((P, CHUNK), jnp.inf, jnp.float32)
        ridx_ref[...] = jnp.zeros((P, CHUNK), jnp.int32)

    cpx = cp_ref[:, 0:1]
    cpy = cp_ref[:, 1:2]
    cpz = cp_ref[:, 2:3]
    cpsq = (cpx * cpx + cpy * cpy) + cpz * cpz
    cpxb = _round_bf16(cpx)
    cpyb = _round_bf16(cpy)
    cpzb = _round_bf16(cpz)
    for k in range(CPB):
        sl = pl.ds(k * CHUNK, CHUNK)
        mx = mu_ref[0:1, sl]
        my = mu_ref[1:2, sl]
        mz = mu_ref[2:3, sl]
        musq = (mx * mx + my * my) + mz * mz
        dot = (cpxb * _round_bf16(mx) + cpyb * _round_bf16(my)) \
            + cpzb * _round_bf16(mz)
        score = (cpsq + musq) - 2.0 * dot
        old = rmin_ref[...]
        upd = score < old
        c = jnp.int32(j * CPB + k)
        rmin_ref[...] = jnp.where(upd, score, old)
        ridx_ref[...] = jnp.where(upd, c, ridx_ref[...])

    @pl.when(j == NSTEPS - 1)
    def _fin():
        rmin = rmin_ref[...]
        m = jnp.min(rmin, axis=1, keepdims=True)
        lane = lax.broadcasted_iota(jnp.int32, (P, CHUNK), 1)
        nf = ridx_ref[...] * CHUNK + lane
        cand = jnp.where(rmin == m, nf, jnp.int32(2**30))
        out_ref[...] = jnp.min(cand, axis=1, keepdims=True)


def _nn_argmin(mu_t, cp):
    return pl.pallas_call(
        _argmin_body,
        grid=(NSTEPS,),
        in_specs=[
            pl.BlockSpec((8, NBLK), lambda j: (0, j)),
            pl.BlockSpec((P, 3), lambda j: (0, 0)),
        ],
        out_specs=pl.BlockSpec((P, 1), lambda j: (0, 0)),
        out_shape=jax.ShapeDtypeStruct((P, 1), jnp.int32),
        scratch_shapes=[
            pltpu.VMEM((P, CHUNK), jnp.float32),
            pltpu.VMEM((P, CHUNK), jnp.int32),
        ],
        compiler_params=pltpu.CompilerParams(
            dimension_semantics=("arbitrary",),
        ),
    )(mu_t, cp)


def _sqrt16(x):
    b = plsc.bitcast(x, jnp.int32)
    y = plsc.bitcast((b >> 1) + jnp.int32(0x1FBD1DF5), jnp.float32)
    for _ in range(3):
        y = 0.5 * (y + x / y)
    return y


def _sc_body(idx_hbm, pos_hbm, scl_hbm, cpf_hbm, conf_hbm, out_hbm,
             idx_v, pos_v, scl_v, cpx_v, cpy_v, cpz_v, w_v, acc_v):
    wid = lax.axis_index("s") * 2 + lax.axis_index("c")
    base = wid * PPW
    pltpu.sync_copy(idx_hbm.at[pl.ds(base, PPW)], idx_v)
    pltpu.sync_copy(pos_hbm, pos_v)
    pltpu.sync_copy(scl_hbm, scl_v)
    pltpu.sync_copy(cpf_hbm.at[pl.ds(0 * P + base, PPW)], cpx_v)
    pltpu.sync_copy(cpf_hbm.at[pl.ds(1 * P + base, PPW)], cpy_v)
    pltpu.sync_copy(cpf_hbm.at[pl.ds(2 * P + base, PPW)], cpz_v)
    pltpu.sync_copy(conf_hbm.at[pl.ds(base, PPW)], w_v)

    acc = jnp.zeros((LANES,), jnp.float32)
    for g in range(GROUPS):
        sl = pl.ds(g * LANES, LANES)
        i3 = idx_v[sl] * 3
        mux = plsc.load_gather(pos_v, [i3])
        muy = plsc.load_gather(pos_v, [i3 + 1])
        muz = plsc.load_gather(pos_v, [i3 + 2])
        scx = plsc.load_gather(scl_v, [i3])
        scy = plsc.load_gather(scl_v, [i3 + 1])
        scz = plsc.load_gather(scl_v, [i3 + 2])
        dx = (cpx_v[sl] - mux) / (jnp.exp(scx) + 1e-6)
        dy = (cpy_v[sl] - muy) / (jnp.exp(scy) + 1e-6)
        dz = (cpz_v[sl] - muz) / (jnp.exp(scz) + 1e-6)
        m2 = dx * dx + dy * dy + dz * dz
        r = _sqrt16(m2) - 1.0
        w = jnp.minimum(jnp.maximum(w_v[sl], 0.0), 1.0)
        acc = acc + r * r * w
    acc_v[...] = acc
    pltpu.sync_copy(acc_v, out_hbm.at[wid])


def _sc_gather_residual(idx, pos_flat, scl_flat, cp_flat, conf):
    mesh = plsc.VectorSubcoreMesh(core_axis_name="c", subcore_axis_name="s")
    fn = functools.partial(
        pl.kernel,
        out_type=jax.ShapeDtypeStruct((NW, LANES), jnp.float32),
        mesh=mesh,
        scratch_types=[
            pltpu.VMEM((PPW,), jnp.int32),
            pltpu.VMEM((3 * N,), jnp.float32),
            pltpu.VMEM((3 * N,), jnp.float32),
            pltpu.VMEM((PPW,), jnp.float32),
            pltpu.VMEM((PPW,), jnp.float32),
            pltpu.VMEM((PPW,), jnp.float32),
            pltpu.VMEM((PPW,), jnp.float32),
            pltpu.VMEM((LANES,), jnp.float32),
        ],
        compiler_params=pltpu.CompilerParams(needs_layout_passes=False),
    )(_sc_body)
    return fn(idx, pos_flat, scl_flat, cp_flat, conf)


def _reduce_body(p_ref, o_ref):
    o_ref[...] = jnp.sum(p_ref[...], axis=(0, 1), keepdims=True) * (1.0 / P)


def _final_mean(partials):
    return pl.pallas_call(
        _reduce_body,
        out_shape=jax.ShapeDtypeStruct((1, 1), jnp.float32),
    )(partials)


def kernel(positions, scales, contact_points, contact_normals, contact_confidence):
    del contact_normals
    mu_t = jnp.concatenate(
        [positions.T, jnp.zeros((5, N), jnp.float32)], axis=0)
    cp_flat = contact_points.T.reshape(3 * P)

    idx = _nn_argmin(mu_t, contact_points).reshape(P)
    partials = _sc_gather_residual(idx, positions.reshape(3 * N),
                                   scales.reshape(3 * N), cp_flat,
                                   contact_confidence)
    return _final_mean(partials).reshape(())


# speedup vs baseline: 1.3805x; 1.3805x over previous
import functools

import jax
import jax.numpy as jnp
from jax import lax
from jax.experimental import pallas as pl
from jax.experimental.pallas import tpu as pltpu
from jax.experimental.pallas import tpu_sc as plsc

N = 16384
P = 4096
NBLK = 1024
NSTEPS = N // NBLK
CHUNK = 128
CPB = NBLK // CHUNK

NW = 32
PPW = P // NW
LANES = 16
GROUPS = PPW // LANES


def _round_bf16(x):
    b = lax.bitcast_convert_type(x, jnp.int32)
    r = (b + jnp.int32(0x7FFF) + ((b >> 16) & jnp.int32(1))) & jnp.int32(
        ~0xFFFF)
    return lax.bitcast_convert_type(r, jnp.float32)


def _argmin_body(ab_ref, mub_ref, mut_ref, cp_ref, out_ref, rmin_ref, ridx_ref):
    j = pl.program_id(0)

    @pl.when(j == 0)
    def _init():
        rmin_ref[...] = jnp.full((P, CHUNK), jnp.inf, jnp.float32)
        ridx_ref[...] = jnp.zeros((P, CHUNK), jnp.int32)

    a2 = ab_ref[...] * jnp.bfloat16(-2.0)
    dotn = lax.dot_general(a2, mub_ref[...], (((1,), (0,)), ((), ())),
                           preferred_element_type=jnp.float32)
    mx = mut_ref[0:1, :]
    my = mut_ref[1:2, :]
    mz = mut_ref[2:3, :]
    musq = (mx * mx + my * my) + mz * mz
    cpx = cp_ref[:, 0:1]
    cpy = cp_ref[:, 1:2]
    cpz = cp_ref[:, 2:3]
    cpsq = (cpx * cpx + cpy * cpy) + cpz * cpz
    score = (cpsq + musq) + dotn

    vals = [score[:, k * CHUNK:(k + 1) * CHUNK] for k in range(CPB)]
    idxs = [jnp.full((P, CHUNK), k, jnp.int32) for k in range(CPB)]
    while len(vals) > 1:
        nv, ni = [], []
        for p in range(0, len(vals), 2):
            av, bv = vals[p], vals[p + 1]
            ai, bi = idxs[p], idxs[p + 1]
            u = bv < av
            nv.append(jnp.where(u, bv, av))
            ni.append(jnp.where(u, bi, ai))
        vals, idxs = nv, ni
    loc, lix = vals[0], idxs[0] + j * CPB
    old = rmin_ref[...]
    u = loc < old
    rmin_ref[...] = jnp.where(u, loc, old)
    ridx_ref[...] = jnp.where(u, lix, ridx_ref[...])

    @pl.when(j == NSTEPS - 1)
    def _fin():
        rmin = rmin_ref[...]
        m = jnp.min(rmin, axis=1, keepdims=True)
        lane = lax.broadcasted_iota(jnp.int32, (P, CHUNK), 1)
        nf = ridx_ref[...] * CHUNK + lane
        cand = jnp.where(rmin == m, nf, jnp.int32(2**30))
        out_ref[...] = jnp.min(cand, axis=1, keepdims=True)


def _nn_argmin(a_b, mu_b, mu_t, cp):
    return pl.pallas_call(
        _argmin_body,
        grid=(NSTEPS,),
        in_specs=[
            pl.BlockSpec((P, 8), lambda j: (0, 0)),
            pl.BlockSpec((8, NBLK), lambda j: (0, j)),
            pl.BlockSpec((8, NBLK), lambda j: (0, j)),
            pl.BlockSpec((P, 3), lambda j: (0, 0)),
        ],
        out_specs=pl.BlockSpec((P, 1), lambda j: (0, 0)),
        out_shape=jax.ShapeDtypeStruct((P, 1), jnp.int32),
        scratch_shapes=[
            pltpu.VMEM((P, CHUNK), jnp.float32),
            pltpu.VMEM((P, CHUNK), jnp.int32),
        ],
        compiler_params=pltpu.CompilerParams(
            dimension_semantics=("arbitrary",),
        ),
    )(a_b, mu_b, mu_t, cp)


def _sqrt16(x):
    b = plsc.bitcast(x, jnp.int32)
    y = plsc.bitcast((b >> 1) + jnp.int32(0x1FBD1DF5), jnp.float32)
    for _ in range(3):
        y = 0.5 * (y + x / y)
    return y


def _sc_body(idx_hbm, pos_hbm, scl_hbm, cpf_hbm, conf_hbm, out_hbm,
             idx_v, pos_v, scl_v, cpx_v, cpy_v, cpz_v, w_v, acc_v):
    wid = lax.axis_index("s") * 2 + lax.axis_index("c")
    base = wid * PPW
    pltpu.sync_copy(idx_hbm.at[pl.ds(base, PPW)], idx_v)
    pltpu.sync_copy(pos_hbm, pos_v)
    pltpu.sync_copy(scl_hbm, scl_v)
    pltpu.sync_copy(cpf_hbm.at[pl.ds(0 * P + base, PPW)], cpx_v)
    pltpu.sync_copy(cpf_hbm.at[pl.ds(1 * P + base, PPW)], cpy_v)
    pltpu.sync_copy(cpf_hbm.at[pl.ds(2 * P + base, PPW)], cpz_v)
    pltpu.sync_copy(conf_hbm.at[pl.ds(base, PPW)], w_v)

    acc = jnp.zeros((LANES,), jnp.float32)
    for g in range(GROUPS):
        sl = pl.ds(g * LANES, LANES)
        i3 = idx_v[sl] * 3
        mux = plsc.load_gather(pos_v, [i3])
        muy = plsc.load_gather(pos_v, [i3 + 1])
        muz = plsc.load_gather(pos_v, [i3 + 2])
        scx = plsc.load_gather(scl_v, [i3])
        scy = plsc.load_gather(scl_v, [i3 + 1])
        scz = plsc.load_gather(scl_v, [i3 + 2])
        dx = (cpx_v[sl] - mux) / (jnp.exp(scx) + 1e-6)
        dy = (cpy_v[sl] - muy) / (jnp.exp(scy) + 1e-6)
        dz = (cpz_v[sl] - muz) / (jnp.exp(scz) + 1e-6)
        m2 = dx * dx + dy * dy + dz * dz
        r = _sqrt16(m2) - 1.0
        w = jnp.minimum(jnp.maximum(w_v[sl], 0.0), 1.0)
        acc = acc + r * r * w
    acc_v[...] = acc
    pltpu.sync_copy(acc_v, out_hbm.at[wid])


def _sc_gather_residual(idx, pos_flat, scl_flat, cp_flat, conf):
    mesh = plsc.VectorSubcoreMesh(core_axis_name="c", subcore_axis_name="s")
    fn = functools.partial(
        pl.kernel,
        out_type=jax.ShapeDtypeStruct((NW, LANES), jnp.float32),
        mesh=mesh,
        scratch_types=[
            pltpu.VMEM((PPW,), jnp.int32),
            pltpu.VMEM((3 * N,), jnp.float32),
            pltpu.VMEM((3 * N,), jnp.float32),
            pltpu.VMEM((PPW,), jnp.float32),
            pltpu.VMEM((PPW,), jnp.float32),
            pltpu.VMEM((PPW,), jnp.float32),
            pltpu.VMEM((PPW,), jnp.float32),
            pltpu.VMEM((LANES,), jnp.float32),
        ],
        compiler_params=pltpu.CompilerParams(needs_layout_passes=False),
    )(_sc_body)
    return fn(idx, pos_flat, scl_flat, cp_flat, conf)


def _reduce_body(p_ref, o_ref):
    o_ref[...] = jnp.sum(p_ref[...], axis=(0, 1), keepdims=True) * (1.0 / P)


def _final_mean(partials):
    return pl.pallas_call(
        _reduce_body,
        out_shape=jax.ShapeDtypeStruct((1, 1), jnp.float32),
    )(partials)


def kernel(positions, scales, contact_points, contact_normals, contact_confidence):
    del contact_normals
    mu_t = jnp.concatenate(
        [positions.T, jnp.zeros((5, N), jnp.float32)], axis=0)
    mu_b = mu_t.astype(jnp.bfloat16)
    a_b = jnp.concatenate(
        [contact_points, jnp.zeros((P, 5), jnp.float32)],
        axis=1).astype(jnp.bfloat16)
    cp_flat = contact_points.T.reshape(3 * P)

    idx = _nn_argmin(a_b, mu_b, mu_t, contact_points).reshape(P)
    partials = _sc_gather_residual(idx, positions.reshape(3 * N),
                                   scales.reshape(3 * N), cp_flat,
                                   contact_confidence)
    return _final_mean(partials).reshape(())


# speedup vs baseline: 1.6578x; 1.2009x over previous
import functools

import jax
import jax.numpy as jnp
from jax import lax
from jax.experimental import pallas as pl
from jax.experimental.pallas import tpu as pltpu
from jax.experimental.pallas import tpu_sc as plsc

N = 16384
P = 4096
NBLK = 1024
NSTEPS = N // NBLK
CHUNK = 128
CPB = NBLK // CHUNK

NW = 32
PPW = P // NW
LANES = 16
GROUPS = PPW // LANES


def _round_bf16(x):
    b = lax.bitcast_convert_type(x, jnp.int32)
    r = (b + jnp.int32(0x7FFF) + ((b >> 16) & jnp.int32(1))) & jnp.int32(
        ~0xFFFF)
    return lax.bitcast_convert_type(r, jnp.float32)


def _argmin_body(ab_ref, mub_ref, mut_ref, cp_ref, out_ref, rmin_ref, ridx_ref):
    j = pl.program_id(0)

    @pl.when(j == 0)
    def _init():
        rmin_ref[...] = jnp.full((P, CHUNK), jnp.inf, jnp.float32)
        ridx_ref[...] = jnp.zeros((P, CHUNK), jnp.int32)

    a2 = ab_ref[...] * jnp.bfloat16(-2.0)
    dotn = lax.dot_general(a2, mub_ref[...], (((1,), (0,)), ((), ())),
                           preferred_element_type=jnp.float32)
    mx = mut_ref[0:1, :]
    my = mut_ref[1:2, :]
    mz = mut_ref[2:3, :]
    musq = (mx * mx + my * my) + mz * mz
    cpx = cp_ref[:, 0:1]
    cpy = cp_ref[:, 1:2]
    cpz = cp_ref[:, 2:3]
    cpsq = (cpx * cpx + cpy * cpy) + cpz * cpz
    score = (cpsq + musq) + dotn

    vals = [score[:, k * CHUNK:(k + 1) * CHUNK] for k in range(CPB)]
    idxs = [jnp.full((P, CHUNK), k, jnp.int32) for k in range(CPB)]
    while len(vals) > 1:
        nv, ni = [], []
        for p in range(0, len(vals), 2):
            av, bv = vals[p], vals[p + 1]
            ai, bi = idxs[p], idxs[p + 1]
            u = bv < av
            nv.append(jnp.where(u, bv, av))
            ni.append(jnp.where(u, bi, ai))
        vals, idxs = nv, ni
    loc, lix = vals[0], idxs[0] + j * CPB
    old = rmin_ref[...]
    u = loc < old
    rmin_ref[...] = jnp.where(u, loc, old)
    ridx_ref[...] = jnp.where(u, lix, ridx_ref[...])

    @pl.when(j == NSTEPS - 1)
    def _fin():
        rmin = rmin_ref[...]
        m = jnp.min(rmin, axis=1, keepdims=True)
        lane = lax.broadcasted_iota(jnp.int32, (P, CHUNK), 1)
        nf = ridx_ref[...] * CHUNK + lane
        cand = jnp.where(rmin == m, nf, jnp.int32(2**30))
        out_ref[...] = jnp.min(cand, axis=1, keepdims=True)


def _nn_argmin(a_b, mu_b, mu_t, cp):
    return pl.pallas_call(
        _argmin_body,
        grid=(NSTEPS,),
        in_specs=[
            pl.BlockSpec((P, 8), lambda j: (0, 0)),
            pl.BlockSpec((8, NBLK), lambda j: (0, j)),
            pl.BlockSpec((8, NBLK), lambda j: (0, j)),
            pl.BlockSpec((P, 3), lambda j: (0, 0)),
        ],
        out_specs=pl.BlockSpec((P, 1), lambda j: (0, 0)),
        out_shape=jax.ShapeDtypeStruct((P, 1), jnp.int32),
        scratch_shapes=[
            pltpu.VMEM((P, CHUNK), jnp.float32),
            pltpu.VMEM((P, CHUNK), jnp.int32),
        ],
        compiler_params=pltpu.CompilerParams(
            dimension_semantics=("arbitrary",),
        ),
    )(a_b, mu_b, mu_t, cp)


def _sqrt16(x):
    b = plsc.bitcast(x, jnp.int32)
    y = plsc.bitcast((b >> 1) + jnp.int32(0x1FBD1DF5), jnp.float32)
    for _ in range(3):
        y = 0.5 * (y + x / y)
    return y


def _sc_body(idx_hbm, tab_hbm, cpf_hbm, conf_hbm, out_hbm,
             idx_v, idb_v, rows_v, cpx_v, cpy_v, cpz_v, w_v, acc_v, red_v,
             shr_v, sem):
    cid = lax.axis_index("c")
    sid = lax.axis_index("s")
    wid = sid * 2 + cid
    base = wid * PPW
    pltpu.sync_copy(idx_hbm.at[pl.ds(base, PPW)], idx_v)
    for g in range(GROUPS):
        sl = pl.ds(g * LANES, LANES)
        idb_v[sl] = idx_v[sl] >> 3
    pltpu.async_copy(tab_hbm.at[idb_v], rows_v, sem).wait()
    pltpu.sync_copy(cpf_hbm.at[pl.ds(0 * P + base, PPW)], cpx_v)
    pltpu.sync_copy(cpf_hbm.at[pl.ds(1 * P + base, PPW)], cpy_v)
    pltpu.sync_copy(cpf_hbm.at[pl.ds(2 * P + base, PPW)], cpz_v)
    pltpu.sync_copy(conf_hbm.at[pl.ds(base, PPW)], w_v)

    acc = jnp.zeros((LANES,), jnp.float32)
    iota = lax.iota(jnp.int32, LANES)
    for g in range(GROUPS):
        sl = pl.ds(g * LANES, LANES)
        rows = iota + jnp.int32(g * LANES)
        inner = (idx_v[sl] & 7) << 4
        mux = plsc.load_gather(rows_v, [rows, inner])
        muy = plsc.load_gather(rows_v, [rows, inner + 1])
        muz = plsc.load_gather(rows_v, [rows, inner + 2])
        scx = plsc.load_gather(rows_v, [rows, inner + 3])
        scy = plsc.load_gather(rows_v, [rows, inner + 4])
        scz = plsc.load_gather(rows_v, [rows, inner + 5])
        dx = (cpx_v[sl] - mux) / (jnp.exp(scx) + 1e-6)
        dy = (cpy_v[sl] - muy) / (jnp.exp(scy) + 1e-6)
        dz = (cpz_v[sl] - muz) / (jnp.exp(scz) + 1e-6)
        m2 = dx * dx + dy * dy + dz * dz
        r = _sqrt16(m2) - 1.0
        w = jnp.minimum(jnp.maximum(w_v[sl], 0.0), 1.0)
        acc = acc + r * r * w

    acc_v[...] = acc
    pltpu.sync_copy(acc_v, shr_v.at[sid + 16])
    plsc.subcore_barrier()

    @pl.when(sid == 0)
    def _reduce():
        pltpu.sync_copy(shr_v.at[pl.ds(16, 16)], red_v)
        s = red_v[0]
        for i in range(1, 16):
            s = s + red_v[i]
        acc_v[...] = plsc.cumsum(s)
        pltpu.sync_copy(acc_v, out_hbm.at[cid])


def _sc_gather_residual(idx, tab, cp_flat, conf):
    mesh = plsc.VectorSubcoreMesh(core_axis_name="c", subcore_axis_name="s")
    fn = functools.partial(
        pl.kernel,
        out_type=jax.ShapeDtypeStruct((2, LANES), jnp.float32),
        mesh=mesh,
        scratch_types=[
            pltpu.VMEM((PPW,), jnp.int32),
            pltpu.VMEM((PPW,), jnp.int32),
            pltpu.VMEM((PPW, 128), jnp.float32),
            pltpu.VMEM((PPW,), jnp.float32),
            pltpu.VMEM((PPW,), jnp.float32),
            pltpu.VMEM((PPW,), jnp.float32),
            pltpu.VMEM((PPW,), jnp.float32),
            pltpu.VMEM((LANES,), jnp.float32),
            pltpu.VMEM((16, LANES), jnp.float32),
            pltpu.VMEM_SHARED((32, LANES), jnp.float32),
            pltpu.SemaphoreType.DMA,
        ],
        compiler_params=pltpu.CompilerParams(needs_layout_passes=False),
    )(_sc_body)
    return fn(idx, tab, cp_flat, conf)


def kernel(positions, scales, contact_points, contact_normals, contact_confidence):
    del contact_normals
    mu_t = jnp.concatenate(
        [positions.T, jnp.zeros((5, N), jnp.float32)], axis=0)
    mu_b = mu_t.astype(jnp.bfloat16)
    a_b = jnp.concatenate(
        [contact_points, jnp.zeros((P, 5), jnp.float32)],
        axis=1).astype(jnp.bfloat16)
    cp_flat = contact_points.T.reshape(3 * P)
    tab = jnp.concatenate(
        [positions, scales, jnp.zeros((N, 10), jnp.float32)],
        axis=1).reshape(N // 8, 128)

    idx = _nn_argmin(a_b, mu_b, mu_t, contact_points).reshape(P)
    partials = _sc_gather_residual(idx, tab, cp_flat, contact_confidence)
    return ((partials[0, 15] + partials[1, 15]) * (1.0 / P)).reshape(())


# speedup vs baseline: 1.8277x; 1.1025x over previous
import functools

import jax
import jax.numpy as jnp
from jax import lax
from jax.experimental import pallas as pl
from jax.experimental.pallas import tpu as pltpu
from jax.experimental.pallas import tpu_sc as plsc

N = 16384
P = 4096
NBLK = 2048
NSTEPS = N // NBLK
CHUNK = 128
CPB = NBLK // CHUNK

NW = 32
PPW = P // NW
LANES = 16
GROUPS = PPW // LANES


def _round_bf16(x):
    b = lax.bitcast_convert_type(x, jnp.int32)
    r = (b + jnp.int32(0x7FFF) + ((b >> 16) & jnp.int32(1))) & jnp.int32(
        ~0xFFFF)
    return lax.bitcast_convert_type(r, jnp.float32)


def _argmin_body(ab_ref, mub_ref, mut_ref, cp_ref, out_ref, rmin_ref, ridx_ref):
    j = pl.program_id(0)

    @pl.when(j == 0)
    def _init():
        rmin_ref[...] = jnp.full((P, CHUNK), jnp.inf, jnp.float32)
        ridx_ref[...] = jnp.zeros((P, CHUNK), jnp.int32)

    a2 = ab_ref[...] * jnp.bfloat16(-2.0)
    dotn = lax.dot_general(a2, mub_ref[...], (((1,), (0,)), ((), ())),
                           preferred_element_type=jnp.float32)
    mx = mut_ref[0:1, :]
    my = mut_ref[1:2, :]
    mz = mut_ref[2:3, :]
    musq = (mx * mx + my * my) + mz * mz
    cpx = cp_ref[:, 0:1]
    cpy = cp_ref[:, 1:2]
    cpz = cp_ref[:, 2:3]
    cpsq = (cpx * cpx + cpy * cpy) + cpz * cpz
    score = (cpsq + musq) + dotn

    vals = [score[:, k * CHUNK:(k + 1) * CHUNK] for k in range(CPB)]
    idxs = [jnp.full((P, CHUNK), k, jnp.int32) for k in range(CPB)]
    while len(vals) > 1:
        nv, ni = [], []
        for p in range(0, len(vals), 2):
            av, bv = vals[p], vals[p + 1]
            ai, bi = idxs[p], idxs[p + 1]
            u = bv < av
            nv.append(jnp.where(u, bv, av))
            ni.append(jnp.where(u, bi, ai))
        vals, idxs = nv, ni
    loc, lix = vals[0], idxs[0] + j * CPB
    old = rmin_ref[...]
    u = loc < old
    rmin_ref[...] = jnp.where(u, loc, old)
    ridx_ref[...] = jnp.where(u, lix, ridx_ref[...])

    @pl.when(j == NSTEPS - 1)
    def _fin():
        rmin = rmin_ref[...]
        m = jnp.min(rmin, axis=1, keepdims=True)
        lane = lax.broadcasted_iota(jnp.int32, (P, CHUNK), 1)
        nf = ridx_ref[...] * CHUNK + lane
        cand = jnp.where(rmin == m, nf, jnp.int32(2**30))
        out_ref[...] = jnp.min(cand, axis=1, keepdims=True)


def _nn_argmin(a_b, mu_b, mu_t, cp):
    return pl.pallas_call(
        _argmin_body,
        grid=(NSTEPS,),
        in_specs=[
            pl.BlockSpec((P, 8), lambda j: (0, 0)),
            pl.BlockSpec((8, NBLK), lambda j: (0, j)),
            pl.BlockSpec((8, NBLK), lambda j: (0, j)),
            pl.BlockSpec((P, 3), lambda j: (0, 0)),
        ],
        out_specs=pl.BlockSpec((P, 1), lambda j: (0, 0)),
        out_shape=jax.ShapeDtypeStruct((P, 1), jnp.int32),
        scratch_shapes=[
            pltpu.VMEM((P, CHUNK), jnp.float32),
            pltpu.VMEM((P, CHUNK), jnp.int32),
        ],
        compiler_params=pltpu.CompilerParams(
            dimension_semantics=("arbitrary",),
        ),
    )(a_b, mu_b, mu_t, cp)


def _sqrt16(x):
    b = plsc.bitcast(x, jnp.int32)
    y = plsc.bitcast((b >> 1) + jnp.int32(0x1FBD1DF5), jnp.float32)
    for _ in range(3):
        y = 0.5 * (y + x / y)
    return y


def _sc_body(idx_hbm, tab_hbm, cpf_hbm, conf_hbm, out_hbm,
             idx_v, idb_v, rows_v, cpx_v, cpy_v, cpz_v, w_v, acc_v, red_v,
             shr_v, sem):
    cid = lax.axis_index("c")
    sid = lax.axis_index("s")
    wid = sid * 2 + cid
    base = wid * PPW
    pltpu.sync_copy(idx_hbm.at[pl.ds(base, PPW)], idx_v)
    for g in range(GROUPS):
        sl = pl.ds(g * LANES, LANES)
        idb_v[sl] = idx_v[sl] >> 3
    pltpu.async_copy(tab_hbm.at[idb_v], rows_v, sem).wait()
    pltpu.sync_copy(cpf_hbm.at[pl.ds(0 * P + base, PPW)], cpx_v)
    pltpu.sync_copy(cpf_hbm.at[pl.ds(1 * P + base, PPW)], cpy_v)
    pltpu.sync_copy(cpf_hbm.at[pl.ds(2 * P + base, PPW)], cpz_v)
    pltpu.sync_copy(conf_hbm.at[pl.ds(base, PPW)], w_v)

    acc = jnp.zeros((LANES,), jnp.float32)
    iota = lax.iota(jnp.int32, LANES)
    for g in range(GROUPS):
        sl = pl.ds(g * LANES, LANES)
        rows = iota + jnp.int32(g * LANES)
        inner = (idx_v[sl] & 7) << 4
        mux = plsc.load_gather(rows_v, [rows, inner])
        muy = plsc.load_gather(rows_v, [rows, inner + 1])
        muz = plsc.load_gather(rows_v, [rows, inner + 2])
        scx = plsc.load_gather(rows_v, [rows, inner + 3])
        scy = plsc.load_gather(rows_v, [rows, inner + 4])
        scz = plsc.load_gather(rows_v, [rows, inner + 5])
        dx = (cpx_v[sl] - mux) / (jnp.exp(scx) + 1e-6)
        dy = (cpy_v[sl] - muy) / (jnp.exp(scy) + 1e-6)
        dz = (cpz_v[sl] - muz) / (jnp.exp(scz) + 1e-6)
        m2 = dx * dx + dy * dy + dz * dz
        r = _sqrt16(m2) - 1.0
        w = jnp.minimum(jnp.maximum(w_v[sl], 0.0), 1.0)
        acc = acc + r * r * w

    acc_v[...] = acc
    pltpu.sync_copy(acc_v, shr_v.at[sid + 16])
    plsc.subcore_barrier()

    @pl.when(sid == 0)
    def _reduce():
        pltpu.sync_copy(shr_v.at[pl.ds(16, 16)], red_v)
        s = red_v[0]
        for i in range(1, 16):
            s = s + red_v[i]
        acc_v[...] = plsc.cumsum(s)
        pltpu.sync_copy(acc_v, out_hbm.at[cid])


def _sc_gather_residual(idx, tab, cp_flat, conf):
    mesh = plsc.VectorSubcoreMesh(core_axis_name="c", subcore_axis_name="s")
    fn = functools.partial(
        pl.kernel,
        out_type=jax.ShapeDtypeStruct((2, LANES), jnp.float32),
        mesh=mesh,
        scratch_types=[
            pltpu.VMEM((PPW,), jnp.int32),
            pltpu.VMEM((PPW,), jnp.int32),
            pltpu.VMEM((PPW, 128), jnp.float32),
            pltpu.VMEM((PPW,), jnp.float32),
            pltpu.VMEM((PPW,), jnp.float32),
            pltpu.VMEM((PPW,), jnp.float32),
            pltpu.VMEM((PPW,), jnp.float32),
            pltpu.VMEM((LANES,), jnp.float32),
            pltpu.VMEM((16, LANES), jnp.float32),
            pltpu.VMEM_SHARED((32, LANES), jnp.float32),
            pltpu.SemaphoreType.DMA,
        ],
        compiler_params=pltpu.CompilerParams(needs_layout_passes=False),
    )(_sc_body)
    return fn(idx, tab, cp_flat, conf)


def kernel(positions, scales, contact_points, contact_normals, contact_confidence):
    del contact_normals
    mu_t = jnp.concatenate(
        [positions.T, jnp.zeros((5, N), jnp.float32)], axis=0)
    mu_b = mu_t.astype(jnp.bfloat16)
    a_b = jnp.concatenate(
        [contact_points, jnp.zeros((P, 5), jnp.float32)],
        axis=1).astype(jnp.bfloat16)
    cp_flat = contact_points.T.reshape(3 * P)
    tab = jnp.concatenate(
        [positions, scales, jnp.zeros((N, 10), jnp.float32)],
        axis=1).reshape(N // 8, 128)

    idx = _nn_argmin(a_b, mu_b, mu_t, contact_points).reshape(P)
    partials = _sc_gather_residual(idx, tab, cp_flat, contact_confidence)
    return ((partials[0, 15] + partials[1, 15]) * (1.0 / P)).reshape(())
